# baseline (speedup 1.0000x reference)
import functools

import jax
import jax.numpy as jnp
from jax import lax
from jax.experimental import pallas as pl
from jax.experimental.pallas import tpu as pltpu
from jax.experimental.pallas import tpu_sc as plsc

_NC, _NS, _L = 2, 16, 16
_W = 512
_NSL = _W // 128


def _stanh(x):
    return 1.0 - 2.0 / (jnp.exp(2.0 * x) + 1.0)


def _rv_logit_body(creq_hbm, cveh_hbm, src_hbm, dst_hbm, a0_hbm, a1_hbm,
                   cst_hbm, logit_hbm,
                   idx_s, idx_d, a0_v, a1_v, cr_buf, cv_buf, out_v, cst_v,
                   sem_a, sem_b):
    c = lax.axis_index("c")
    s = lax.axis_index("s")
    wid = c * _NS + s
    NW = _NC * _NS
    E = src_hbm.shape[0] * 128
    n_chunks = E // _W
    my_n = (n_chunks + NW - 1 - wid) // NW

    pltpu.sync_copy(cst_hbm, cst_v)
    u0 = [cst_v[0, pl.ds(h * _L, _L)] for h in range(6)]
    u1 = [cst_v[1, pl.ds(h * _L, _L)] for h in range(6)]
    w2 = [cst_v[2, pl.ds(h * _L, _L)] for h in range(6)]
    batt = cst_v[3, pl.ds(0, _L)][0]
    iota = lax.iota(jnp.int32, _L)

    def _chunk(k, _):
        g = wid + NW * k
        r0 = g * _NSL
        pltpu.sync_copy(src_hbm.at[pl.ds(r0, _NSL)], idx_s)
        pltpu.sync_copy(dst_hbm.at[pl.ds(r0, _NSL)], idx_d)
        pltpu.sync_copy(a0_hbm.at[pl.ds(r0, _NSL)], a0_v)
        pltpu.sync_copy(a1_hbm.at[pl.ds(r0, _NSL)], a1_v)
        cps = ([pltpu.make_async_copy(
                    creq_hbm.at[idx_s.at[j]],
                    cr_buf.at[pl.ds(j * 128, 128)], sem_a)
                for j in range(_NSL)]
               + [pltpu.make_async_copy(
                    cveh_hbm.at[idx_d.at[j]],
                    cv_buf.at[pl.ds(j * 128, 128)], sem_b)
                  for j in range(_NSL)])
        for cp in cps:
            cp.start()
        for cp in cps:
            cp.wait()

        def _grp(gi, _):
            av0 = a0_v[gi // 8, pl.ds((gi % 8) * _L, _L)]
            av1 = a1_v[gi // 8, pl.ds((gi % 8) * _L, _L)]
            res = jnp.zeros((_L,), jnp.float32)
            for j2 in range(_L):
                j = gi * _L + j2
                s0 = av0[j2]
                s1 = av1[j2]
                acc = jnp.zeros((_L,), jnp.float32)
                for h in range(6):
                    z = (cr_buf[j, pl.ds(h * _L, _L)]
                         + cv_buf[j, pl.ds(h * _L, _L)]
                         + s0 * u0[h] + s1 * u1[h])
                    acc = acc + _stanh(z) * w2[h]
                lg = batt
                for kl in range(_L):
                    lg = lg + acc[kl]
                res = jnp.where(iota == j2, jnp.full((_L,), lg), res)
            out_v[gi // 8, pl.ds((gi % 8) * _L, _L)] = res
            return 0
        lax.fori_loop(0, _W // _L, _grp, 0)
        pltpu.sync_copy(out_v, logit_hbm.at[pl.ds(r0, _NSL)])
        return 0

    lax.fori_loop(0, my_n, _chunk, 0)


def _run_rv_logits(Creq, Cveh, src, dst, a0, a1, cst):
    E = src.shape[0]
    mesh = plsc.VectorSubcoreMesh(core_axis_name="c", subcore_axis_name="s")
    f = functools.partial(
        pl.kernel,
        out_type=jax.ShapeDtypeStruct((E // 128, 128), jnp.float32),
        mesh=mesh,
        scratch_types=[
            pltpu.VMEM((_NSL, 128), jnp.int32),
            pltpu.VMEM((_NSL, 128), jnp.int32),
            pltpu.VMEM((_NSL, 128), jnp.float32),
            pltpu.VMEM((_NSL, 128), jnp.float32),
            pltpu.VMEM((_W, 96), jnp.float32),
            pltpu.VMEM((_W, 96), jnp.float32),
            pltpu.VMEM((_NSL, 128), jnp.float32),
            pltpu.VMEM((4, 96), jnp.float32),
            pltpu.SemaphoreType.DMA,
            pltpu.SemaphoreType.DMA,
        ],
        compiler_params=pltpu.CompilerParams(use_tc_tiling_on_sc=False),
    )(_rv_logit_body)
    return f(Creq, Cveh, src.reshape(-1, 128), dst.reshape(-1, 128),
             a0.reshape(-1, 128), a1.reshape(-1, 128), cst)


def _rv_acc_body(vfA_hbm, vfB_hbm, lg_hbm, src_hbm, dst_hbm, gmax_hbm,
                 nsumA_hbm, nsumB_hbm, d_hbm, cnt_hbm,
                 idx_s, idx_d, loc, lg_v, e_v, vf_buf, ones_v, gmax_v,
                 zero_buf, zero1_v, acc_n, acc_d, acc_c, sem_a, sem_w):
    c = lax.axis_index("c")
    s = lax.axis_index("s")
    R = d_hbm.shape[0]
    Rq = R // 4
    ACC = acc_n.shape[0]
    E = src_hbm.shape[0] * 128

    _fill(zero_buf, jnp.zeros((_L,), jnp.float32))
    _fill(zero1_v, jnp.zeros((_L,), jnp.float32))
    _fill(ones_v, jnp.ones((_L,), jnp.float32))
    pltpu.sync_copy(gmax_hbm, gmax_v)
    gmax = gmax_v[pl.ds(0, _L)][0]

    rows_per_tile = ACC // _NS
    n_chunks = E // _W
    my_n = (n_chunks + _NS - 1 - s) // _NS
    iota = lax.iota(jnp.int32, _L)

    for q in range(2):
        base = (c * 2 + q) * Rq
        for fh in range(2):
            vf_hbm = vfA_hbm if fh == 0 else vfB_hbm
            nout_hbm = nsumA_hbm if fh == 0 else nsumB_hbm

            for z in range(rows_per_tile // zero_buf.shape[0]):
                pltpu.sync_copy(
                    zero_buf,
                    acc_n.at[pl.ds(s * rows_per_tile + z * zero_buf.shape[0],
                                   zero_buf.shape[0])])
            if fh == 0:
                pltpu.sync_copy(
                    zero1_v.at[pl.ds(0, rows_per_tile)],
                    acc_d.at[pl.ds(s * rows_per_tile, rows_per_tile)])
                pltpu.sync_copy(
                    zero1_v.at[pl.ds(0, rows_per_tile)],
                    acc_c.at[pl.ds(s * rows_per_tile, rows_per_tile)])
            plsc.subcore_barrier()

            def _chunk(k, _):
                g = s + _NS * k
                r0 = g * _NSL
                pltpu.sync_copy(src_hbm.at[pl.ds(r0, _NSL)], idx_s)
                pltpu.sync_copy(dst_hbm.at[pl.ds(r0, _NSL)], idx_d)
                pltpu.sync_copy(lg_hbm.at[pl.ds(r0, _NSL)], lg_v)
                cps = [pltpu.make_async_copy(
                    vf_hbm.at[idx_d.at[j]],
                    vf_buf.at[pl.ds(j * 128, 128)], sem_a)
                    for j in range(_NSL)]
                for cp in cps:
                    cp.start()

                def _locs(i, _):
                    sv = idx_s[i // 8, pl.ds((i % 8) * _L, _L)]
                    lo = sv - base
                    inb = (lo >= 0) & (lo < Rq)
                    trash = Rq + ((i * _L + iota) & 511)
                    loc[i // 8, pl.ds((i % 8) * _L, _L)] = (
                        jnp.where(inb, lo, trash))
                    ev = jnp.exp(lg_v[i // 8, pl.ds((i % 8) * _L, _L)] - gmax)
                    e_v[i // 8, pl.ds((i % 8) * _L, _L)] = ev
                    return 0
                lax.fori_loop(0, _W // _L, _locs, 0)
                for cp in cps:
                    cp.wait()

                def _rows(gi, _):
                    ev16 = e_v[gi // 8, pl.ds((gi % 8) * _L, _L)]
                    for j2 in range(_L):
                        j = gi * _L + j2
                        e = ev16[j2]
                        for h in range(2):
                            vf_buf[j, pl.ds(h * _L, _L)] = (
                                vf_buf[j, pl.ds(h * _L, _L)] * e)
                    return 0
                lax.fori_loop(0, _W // _L, _rows, 0)

                for j in range(_NSL):
                    pltpu.sync_copy(vf_buf.at[pl.ds(j * 128, 128)],
                                    acc_n.at[loc.at[j]], add=True)
                    if fh == 0:
                        pltpu.sync_copy(e_v.at[j], acc_d.at[loc.at[j]],
                                        add=True)
                        pltpu.sync_copy(ones_v.at[pl.ds(j * 128, 128)],
                                        acc_c.at[loc.at[j]], add=True)
                return 0

            lax.fori_loop(0, my_n, _chunk, 0)
            plsc.subcore_barrier()

            r10 = Rq // 10
            r5 = Rq // 5

            @pl.when(s < 10)
            def _():
                pltpu.sync_copy(acc_n.at[pl.ds(s * r10, r10)],
                                nout_hbm.at[pl.ds(base + s * r10, r10)])

            if fh == 0:
                @pl.when(s < 5)
                def _():
                    pltpu.sync_copy(acc_d.at[pl.ds(s * r5, r5)],
                                    d_hbm.at[pl.ds(base + s * r5, r5)])
                    pltpu.sync_copy(acc_c.at[pl.ds(s * r5, r5)],
                                    cnt_hbm.at[pl.ds(base + s * r5, r5)])
            plsc.subcore_barrier()


def _run_rv_acc(vfA, vfB, logits, src, dst, gmax16, R):
    E = src.shape[0]
    acc_rows = R // 4 + 512
    acc_rows += (-acc_rows) % (_NS * 800)
    mesh = plsc.VectorSubcoreMesh(core_axis_name="c", subcore_axis_name="s")
    f = functools.partial(
        pl.kernel,
        out_type=[jax.ShapeDtypeStruct((R, 32), jnp.float32),
                  jax.ShapeDtypeStruct((R, 32), jnp.float32),
                  jax.ShapeDtypeStruct((R,), jnp.float32),
                  jax.ShapeDtypeStruct((R,), jnp.float32)],
        mesh=mesh,
        scratch_types=[
            pltpu.VMEM((_NSL, 128), jnp.int32),
            pltpu.VMEM((_NSL, 128), jnp.int32),
            pltpu.VMEM((_NSL, 128), jnp.int32),
            pltpu.VMEM((_NSL, 128), jnp.float32),
            pltpu.VMEM((_NSL, 128), jnp.float32),
            pltpu.VMEM((_W, 32), jnp.float32),
            pltpu.VMEM((_W,), jnp.float32),
            pltpu.VMEM((16,), jnp.float32),
            pltpu.VMEM((800, 32), jnp.float32),
            pltpu.VMEM((acc_rows // _NS,), jnp.float32),
            pltpu.VMEM_SHARED((acc_rows, 32), jnp.float32),
            pltpu.VMEM_SHARED((acc_rows,), jnp.float32),
            pltpu.VMEM_SHARED((acc_rows,), jnp.float32),
            pltpu.SemaphoreType.DMA,
            pltpu.SemaphoreType.DMA,
        ],
        compiler_params=pltpu.CompilerParams(use_tc_tiling_on_sc=False),
    )(_rv_acc_body)
    return f(vfA, vfB, logits, src.reshape(-1, 128), dst.reshape(-1, 128),
             gmax16)


def _fill(ref, vec):
    if len(ref.shape) == 1:
        def _f(i, _):
            ref[pl.ds(i * _L, _L)] = vec
            return 0
        lax.fori_loop(0, ref.shape[0] // _L, _f, 0)
    else:
        hs = ref.shape[1] // _L

        def _f(i, _):
            ref[i // hs, pl.ds((i % hs) * _L, _L)] = vec
            return 0
        lax.fori_loop(0, ref.shape[0] * hs, _f, 0)


def _rr_body(A_hbm, B_hbm, src_hbm, dst_hbm, attr_hbm, wb_hbm,
             esum_hbm, ecnt_hbm,
             idx_s, idx_d, loc, attr_v, a_buf, b_buf, ones_v, wb_v,
             zero_buf, zero1_v, acc_sh, cnt_sh, sem_a, sem_b, sem_w):
    c = lax.axis_index("c")
    s = lax.axis_index("s")
    R = A_hbm.shape[0]
    Rq = R // 4
    ACC = acc_sh.shape[0]
    E = src_hbm.shape[0] * 128

    _fill(zero_buf, jnp.zeros((_L,), jnp.float32))
    _fill(zero1_v, jnp.zeros((_L,), jnp.float32))
    _fill(ones_v, jnp.ones((_L,), jnp.float32))
    pltpu.sync_copy(wb_hbm, wb_v)
    wv = [wb_v[0, pl.ds(h * _L, _L)] for h in range(2)]
    bb = [wb_v[1, pl.ds(h * _L, _L)] for h in range(2)]

    rows_per_tile = ACC // _NS
    n_chunks = (E // _W)
    my_n = (n_chunks + _NS - 1 - s) // _NS
    iota = lax.iota(jnp.int32, _L)

    for q in range(2):
        base = (c * 2 + q) * Rq

        for z in range(rows_per_tile // zero_buf.shape[0]):
            pltpu.sync_copy(
                zero_buf,
                acc_sh.at[pl.ds(s * rows_per_tile + z * zero_buf.shape[0],
                                zero_buf.shape[0])])
        pltpu.sync_copy(zero1_v.at[pl.ds(0, rows_per_tile)],
                        cnt_sh.at[pl.ds(s * rows_per_tile, rows_per_tile)])
        plsc.subcore_barrier()

        def _chunk(k, _):
            g = s + _NS * k
            r0 = g * _NSL
            pltpu.sync_copy(src_hbm.at[pl.ds(r0, _NSL)], idx_s)
            pltpu.sync_copy(dst_hbm.at[pl.ds(r0, _NSL)], idx_d)
            pltpu.sync_copy(attr_hbm.at[pl.ds(r0, _NSL)], attr_v)
            cp_a = [pltpu.make_async_copy(
                A_hbm.at[idx_s.at[j]], a_buf.at[pl.ds(j * 128, 128)], sem_a)
                for j in range(_NSL)]
            cp_b = [pltpu.make_async_copy(
                B_hbm.at[idx_d.at[j]], b_buf.at[pl.ds(j * 128, 128)], sem_b)
                for j in range(_NSL)]
            for cp in cp_a + cp_b:
                cp.start()
            for cp in cp_a + cp_b:
                cp.wait()

            def _locs(i, _):
                sv = idx_s[i // 8, pl.ds((i % 8) * _L, _L)]
                lo = sv - base
                inb = (lo >= 0) & (lo < Rq)
                trash = Rq + ((i * _L + iota) & 511)
                safe = jnp.where(inb, lo, trash)
                loc[i // 8, pl.ds((i % 8) * _L, _L)] = safe
                return 0
            lax.fori_loop(0, _W // _L, _locs, 0)

            def _edge(gi, _):
                av16 = attr_v[gi // 8, pl.ds((gi % 8) * _L, _L)]
                for j2 in range(_L):
                    j = gi * _L + j2
                    av = av16[j2]
                    for h in range(2):
                        va = a_buf[j, pl.ds(h * _L, _L)]
                        vb = b_buf[j, pl.ds(h * _L, _L)]
                        a_buf[j, pl.ds(h * _L, _L)] = _stanh(
                            va + vb + av * wv[h] + bb[h])
                return 0
            lax.fori_loop(0, _W // _L, _edge, 0)

            for j in range(_NSL):
                pltpu.sync_copy(a_buf.at[pl.ds(j * 128, 128)],
                                acc_sh.at[loc.at[j]], add=True)
                pltpu.sync_copy(ones_v.at[pl.ds(j * 128, 128)],
                                cnt_sh.at[loc.at[j]], add=True)
            return 0

        lax.fori_loop(0, my_n, _chunk, 0)
        plsc.subcore_barrier()

        r10 = Rq // 10
        r5 = Rq // 5

        @pl.when(s < 10)
        def _():
            pltpu.sync_copy(acc_sh.at[pl.ds(s * r10, r10)],
                            esum_hbm.at[pl.ds(base + s * r10, r10)])

        @pl.when(s < 5)
        def _():
            pltpu.sync_copy(cnt_sh.at[pl.ds(s * r5, r5)],
                            ecnt_hbm.at[pl.ds(base + s * r5, r5)])
        plsc.subcore_barrier()


def _run_rr(A, B, src, dst, attr, wb):
    R = A.shape[0]
    acc_rows = R // 4 + 512
    acc_rows += (-acc_rows) % (_NS * 800)
    mesh = plsc.VectorSubcoreMesh(core_axis_name="c", subcore_axis_name="s")
    f = functools.partial(
        pl.kernel,
        out_type=[jax.ShapeDtypeStruct((R, 32), jnp.float32),
                  jax.ShapeDtypeStruct((R,), jnp.float32)],
        mesh=mesh,
        scratch_types=[
            pltpu.VMEM((_NSL, 128), jnp.int32),
            pltpu.VMEM((_NSL, 128), jnp.int32),
            pltpu.VMEM((_NSL, 128), jnp.int32),
            pltpu.VMEM((_NSL, 128), jnp.float32),
            pltpu.VMEM((_W, 32), jnp.float32),
            pltpu.VMEM((_W, 32), jnp.float32),
            pltpu.VMEM((_W,), jnp.float32),
            pltpu.VMEM((2, 32), jnp.float32),
            pltpu.VMEM((800, 32), jnp.float32),
            pltpu.VMEM((acc_rows // _NS, ), jnp.float32),
            pltpu.VMEM_SHARED((acc_rows, 32), jnp.float32),
            pltpu.VMEM_SHARED((acc_rows,), jnp.float32),
            pltpu.SemaphoreType.DMA,
            pltpu.SemaphoreType.DMA,
            pltpu.SemaphoreType.DMA,
        ],
        compiler_params=pltpu.CompilerParams(use_tc_tiling_on_sc=False),
    )(_rr_body)
    return f(A, B, src.reshape(-1, 128), dst.reshape(-1, 128),
             attr.reshape(-1, 128), wb)


_BR = 2000


def _head_body(reqx_ref, esum_ref, ecnt_ref, nsum_ref, d_ref, cnt_ref,
               wr_ref, br_ref, wn1_ref, wn2_ref, bn_ref,
               w1_ref, b1_ref, w2_ref, b2_ref, w3_ref, b3_ref, out_ref):
    reqx = reqx_ref[...]
    req_feat = jnp.dot(reqx, wr_ref[...],
                       preferred_element_type=jnp.float32) + br_ref[...]
    edge_mean = esum_ref[...] / jnp.maximum(ecnt_ref[...], 1.0)
    trip = (jnp.dot(reqx, wn1_ref[...], preferred_element_type=jnp.float32)
            + jnp.dot(edge_mean, wn2_ref[...],
                      preferred_element_type=jnp.float32) + bn_ref[...])
    veh_agg = nsum_ref[...] / (jnp.maximum(d_ref[...], 1e-30)
                               * jnp.maximum(cnt_ref[...], 1.0))
    act = jnp.concatenate([req_feat, trip, veh_agg], axis=1)
    h = jnp.tanh(jnp.dot(act, w1_ref[...],
                         preferred_element_type=jnp.float32) + b1_ref[...])
    h = jnp.tanh(jnp.dot(h, w2_ref[...],
                         preferred_element_type=jnp.float32) + b2_ref[...])
    out_ref[...] = jnp.dot(h, w3_ref[...],
                           preferred_element_type=jnp.float32) + b3_ref[...]


def _run_head(reqx, esum, ecnt, nsum, d, cnt,
              W_req, b_req, Wn1, Wn2, bn, W1, b1, W2, b2, W3, b3):
    R = reqx.shape[0]
    grid = (R // _BR,)
    row = lambda i: (i, 0)
    full = lambda i: (0, 0)
    return pl.pallas_call(
        _head_body,
        grid=grid,
        in_specs=[
            pl.BlockSpec((_BR, 8), row),
            pl.BlockSpec((_BR, 32), row),
            pl.BlockSpec((_BR, 1), row),
            pl.BlockSpec((_BR, 64), row),
            pl.BlockSpec((_BR, 1), row),
            pl.BlockSpec((_BR, 1), row),
            pl.BlockSpec((8, 32), full),
            pl.BlockSpec((1, 32), full),
            pl.BlockSpec((8, 32), full),
            pl.BlockSpec((32, 32), full),
            pl.BlockSpec((1, 32), full),
            pl.BlockSpec((128, 512), full),
            pl.BlockSpec((1, 512), full),
            pl.BlockSpec((512, 512), full),
            pl.BlockSpec((1, 512), full),
            pl.BlockSpec((512, 1), full),
            pl.BlockSpec((1, 1), full),
        ],
        out_specs=pl.BlockSpec((_BR, 1), row),
        out_shape=jax.ShapeDtypeStruct((R, 1), jnp.float32),
    )(reqx, esum, ecnt, nsum, d, cnt,
      W_req, b_req.reshape(1, -1), Wn1, Wn2, bn.reshape(1, -1),
      W1, b1.reshape(1, -1), W2, b2.reshape(1, -1), W3, b3.reshape(1, -1))


def kernel(passengers_x, vehicles_x, requests_x,
           veh2pas_sender_edge_index, veh2pas_receiver_edge_index,
           req2req_edge_index, req2req_edge_attr,
           req2veh_sender_edge_index, req2veh_receiver_edge_index,
           req2veh_edge_attr,
           W_pas, b_pas, W_veh, b_veh, W_req, b_req,
           W_trip_e, b_trip_e, W_trip_n, b_trip_n, W_rv, b_rv,
           W_att1, b_att1, W_att2, b_att2,
           W_act1, b_act1, W_act2, b_act2, W_act3, b_act3):
    R = requests_x.shape[0]
    V = vehicles_x.shape[0]

    pas_feat = passengers_x @ W_pas + b_pas
    g = pas_feat[veh2pas_receiver_edge_index]
    vp_sum = jax.ops.segment_sum(g, veh2pas_sender_edge_index, num_segments=V)
    vp_cnt = jax.ops.segment_sum(
        jnp.ones((g.shape[0], 1), jnp.float32),
        veh2pas_sender_edge_index, num_segments=V)
    pas_mean = vp_sum / jnp.maximum(vp_cnt, 1.0)
    veh_feat = jnp.concatenate(
        [vehicles_x @ W_veh + b_veh, pas_mean], axis=1)

    A = requests_x @ W_trip_e[:8]
    B = requests_x @ W_trip_e[8:16]
    w_attr = W_trip_e[16]
    src_rr = req2req_edge_index[0]
    dst_rr = req2req_edge_index[1]
    wb = jnp.stack([w_attr, b_trip_e])
    esum, ecnt1 = _run_rr(A, B, src_rr, dst_rr,
                          req2req_edge_attr.reshape(-1), wb)
    ecnt = ecnt1.reshape(-1, 1)

    src_rv = req2veh_sender_edge_index
    dst_rv = req2veh_receiver_edge_index
    Creq = requests_x @ (W_rv[:8] @ W_att1[64:96])
    Cveh = (veh_feat @ W_att1[:64]
            + (vehicles_x @ W_rv[8:13] + b_rv) @ W_att1[64:96]
            + b_att1)
    U = W_rv[13:15] @ W_att1[64:96]
    cst = jnp.zeros((4, 96), jnp.float32)
    cst = cst.at[0].set(U[0]).at[1].set(U[1])
    cst = cst.at[2].set(W_att2[:, 0]).at[3, 0].set(b_att2[0])
    logits = _run_rv_logits(Creq, Cveh, src_rv, dst_rv,
                            req2veh_edge_attr[:, 0], req2veh_edge_attr[:, 1],
                            cst)
    gmax = jnp.max(logits)
    gmax16 = jnp.full((16,), gmax, jnp.float32)
    vfA = veh_feat[:, :32]
    vfB = veh_feat[:, 32:]
    nsumA, nsumB, dsum1, rvcnt1 = _run_rv_acc(
        vfA, vfB, logits, src_rv, dst_rv, gmax16, R)
    nsum = jnp.concatenate([nsumA, nsumB], axis=1)
    dsum = dsum1.reshape(-1, 1)
    rvcnt = rvcnt1.reshape(-1, 1)

    Wn1 = W_trip_n[:8]
    Wn2 = W_trip_n[8:40]
    return _run_head(requests_x, esum, ecnt, nsum, dsum, rvcnt,
                     W_req, b_req, Wn1, Wn2, b_trip_n,
                     W_act1, b_act1, W_act2, b_act2, W_act3, b_act3)


# speedup vs baseline: 1.3369x; 1.3369x over previous
import functools

import jax
import jax.numpy as jnp
from jax import lax
from jax.experimental import pallas as pl
from jax.experimental.pallas import tpu as pltpu
from jax.experimental.pallas import tpu_sc as plsc

_NC, _NS, _L = 2, 16, 16
_W = 512
_NSL = _W // 128


def _stanh(x):
    return 1.0 - 2.0 / (jnp.exp(2.0 * x) + 1.0)


def _rv_logit_body(creq_hbm, cveh_hbm, src_hbm, dst_hbm, a0_hbm, a1_hbm,
                   cst_hbm, logit_hbm,
                   idx_s, idx_d, a0_v, a1_v, cr_buf, cv_buf, out_v, cst_v,
                   sem_a, sem_b):
    c = lax.axis_index("c")
    s = lax.axis_index("s")
    wid = c * _NS + s
    NW = _NC * _NS
    E = src_hbm.shape[0]
    n_chunks = E // _W
    my_n = (n_chunks + NW - 1 - wid) // NW

    pltpu.sync_copy(cst_hbm, cst_v)
    u0 = [cst_v[0, pl.ds(h * _L, _L)] for h in range(6)]
    u1 = [cst_v[1, pl.ds(h * _L, _L)] for h in range(6)]
    w2 = [cst_v[2, pl.ds(h * _L, _L)] for h in range(6)]
    batt = cst_v[3, pl.ds(0, _L)][0]
    iota = lax.iota(jnp.int32, _L)

    def _chunk(k, _):
        g = wid + NW * k
        e0 = g * _W
        for j in range(_NSL):
            pltpu.sync_copy(src_hbm.at[pl.ds(e0 + j * 128, 128)], idx_s.at[j])
            pltpu.sync_copy(dst_hbm.at[pl.ds(e0 + j * 128, 128)], idx_d.at[j])
        pltpu.sync_copy(a0_hbm.at[pl.ds(e0, _W)], a0_v)
        pltpu.sync_copy(a1_hbm.at[pl.ds(e0, _W)], a1_v)
        cps = ([pltpu.make_async_copy(
                    creq_hbm.at[idx_s.at[j]],
                    cr_buf.at[pl.ds(j * 128, 128)], sem_a)
                for j in range(_NSL)]
               + [pltpu.make_async_copy(
                    cveh_hbm.at[idx_d.at[j]],
                    cv_buf.at[pl.ds(j * 128, 128)], sem_b)
                  for j in range(_NSL)])
        for cp in cps:
            cp.start()
        for cp in cps:
            cp.wait()

        def _grp(gi, _):
            av0 = a0_v[pl.ds(gi * _L, _L)]
            av1 = a1_v[pl.ds(gi * _L, _L)]
            res = jnp.zeros((_L,), jnp.float32)
            for j2 in range(_L):
                j = gi * _L + j2
                s0 = av0[j2]
                s1 = av1[j2]
                acc = jnp.zeros((_L,), jnp.float32)
                for h in range(6):
                    z = (cr_buf[j, pl.ds(h * _L, _L)]
                         + cv_buf[j, pl.ds(h * _L, _L)]
                         + s0 * u0[h] + s1 * u1[h])
                    acc = acc + _stanh(z) * w2[h]
                lg = batt
                for kl in range(_L):
                    lg = lg + acc[kl]
                res = jnp.where(iota == j2, jnp.full((_L,), lg), res)
            out_v[pl.ds(gi * _L, _L)] = res
            return 0
        lax.fori_loop(0, _W // _L, _grp, 0)
        pltpu.sync_copy(out_v, logit_hbm.at[pl.ds(e0, _W)])
        return 0

    lax.fori_loop(0, my_n, _chunk, 0)


def _run_rv_logits(Creq, Cveh, src, dst, a0, a1, cst):
    E = src.shape[0]
    mesh = plsc.VectorSubcoreMesh(core_axis_name="c", subcore_axis_name="s")
    f = functools.partial(
        pl.kernel,
        out_type=jax.ShapeDtypeStruct((E,), jnp.float32),
        mesh=mesh,
        scratch_types=[
            pltpu.VMEM((_NSL, 128), jnp.int32),
            pltpu.VMEM((_NSL, 128), jnp.int32),
            pltpu.VMEM((_W,), jnp.float32),
            pltpu.VMEM((_W,), jnp.float32),
            pltpu.VMEM((_W, 96), jnp.float32),
            pltpu.VMEM((_W, 96), jnp.float32),
            pltpu.VMEM((_W,), jnp.float32),
            pltpu.VMEM((4, 96), jnp.float32),
            pltpu.SemaphoreType.DMA,
            pltpu.SemaphoreType.DMA,
        ],
        compiler_params=pltpu.CompilerParams(use_tc_tiling_on_sc=False),
    )(_rv_logit_body)
    return f(Creq, Cveh, src, dst, a0, a1, cst)


def _rv_acc_body(vfA_hbm, vfB_hbm, lg_hbm, src_hbm, dst_hbm, gmax_hbm,
                 nsumA_hbm, nsumB_hbm, d_hbm, cnt_hbm,
                 idx_s, idx_d, loc, lg_v, e_v, vf_buf, ones_v, gmax_v,
                 zero_buf, zero1_v, acc_n, acc_d, acc_c, sem_a, sem_w):
    c = lax.axis_index("c")
    s = lax.axis_index("s")
    R = d_hbm.shape[0]
    Rq = R // 4
    ACC = acc_n.shape[0]
    E = src_hbm.shape[0]

    _fill(zero_buf, jnp.zeros((_L,), jnp.float32))
    _fill(zero1_v, jnp.zeros((_L,), jnp.float32))
    _fill(ones_v, jnp.ones((_L,), jnp.float32))
    pltpu.sync_copy(gmax_hbm, gmax_v)
    gmax = gmax_v[pl.ds(0, _L)][0]

    rows_per_tile = ACC // _NS
    n_chunks = E // _W
    my_n = (n_chunks + _NS - 1 - s) // _NS
    iota = lax.iota(jnp.int32, _L)

    for q in range(2):
        base = (c * 2 + q) * Rq
        for fh in range(2):
            vf_hbm = vfA_hbm if fh == 0 else vfB_hbm
            nout_hbm = nsumA_hbm if fh == 0 else nsumB_hbm

            for z in range(rows_per_tile // zero_buf.shape[0]):
                pltpu.sync_copy(
                    zero_buf,
                    acc_n.at[pl.ds(s * rows_per_tile + z * zero_buf.shape[0],
                                   zero_buf.shape[0])])
            if fh == 0:
                pltpu.sync_copy(
                    zero1_v.at[pl.ds(0, rows_per_tile)],
                    acc_d.at[pl.ds(s * rows_per_tile, rows_per_tile)])
                pltpu.sync_copy(
                    zero1_v.at[pl.ds(0, rows_per_tile)],
                    acc_c.at[pl.ds(s * rows_per_tile, rows_per_tile)])
            plsc.subcore_barrier()

            def _chunk(k, _):
                g = s + _NS * k
                e0 = g * _W
                for j in range(_NSL):
                    pltpu.sync_copy(src_hbm.at[pl.ds(e0 + j * 128, 128)],
                                    idx_s.at[j])
                    pltpu.sync_copy(dst_hbm.at[pl.ds(e0 + j * 128, 128)],
                                    idx_d.at[j])
                pltpu.sync_copy(lg_hbm.at[pl.ds(e0, _W)], lg_v)
                cps = [pltpu.make_async_copy(
                    vf_hbm.at[idx_d.at[j]],
                    vf_buf.at[pl.ds(j * 128, 128)], sem_a)
                    for j in range(_NSL)]
                for cp in cps:
                    cp.start()

                def _locs(i, _):
                    sv = idx_s[i // 8, pl.ds((i % 8) * _L, _L)]
                    lo = sv - base
                    inb = (lo >= 0) & (lo < Rq)
                    trash = Rq + ((i * _L + iota) & 511)
                    loc[i // 8, pl.ds((i % 8) * _L, _L)] = (
                        jnp.where(inb, lo, trash))
                    ev = jnp.exp(lg_v[pl.ds(i * _L, _L)] - gmax)
                    e_v[pl.ds(i * _L, _L)] = ev
                    return 0
                lax.fori_loop(0, _W // _L, _locs, 0)
                for cp in cps:
                    cp.wait()

                def _rows(gi, _):
                    ev16 = e_v[pl.ds(gi * _L, _L)]
                    for j2 in range(_L):
                        j = gi * _L + j2
                        e = ev16[j2]
                        for h in range(2):
                            vf_buf[j, pl.ds(h * _L, _L)] = (
                                vf_buf[j, pl.ds(h * _L, _L)] * e)
                    return 0
                lax.fori_loop(0, _W // _L, _rows, 0)

                for j in range(_NSL):
                    pltpu.sync_copy(vf_buf.at[pl.ds(j * 128, 128)],
                                    acc_n.at[loc.at[j]], add=True)
                    if fh == 0:
                        pltpu.sync_copy(e_v.at[pl.ds(j * 128, 128)],
                                        acc_d.at[loc.at[j]], add=True)
                        pltpu.sync_copy(ones_v.at[pl.ds(j * 128, 128)],
                                        acc_c.at[loc.at[j]], add=True)
                return 0

            lax.fori_loop(0, my_n, _chunk, 0)
            plsc.subcore_barrier()

            r10 = Rq // 10
            r5 = Rq // 5

            @pl.when(s < 10)
            def _():
                pltpu.sync_copy(acc_n.at[pl.ds(s * r10, r10)],
                                nout_hbm.at[pl.ds(base + s * r10, r10)])

            if fh == 0:
                @pl.when(s < 5)
                def _():
                    pltpu.sync_copy(acc_d.at[pl.ds(s * r5, r5)],
                                    d_hbm.at[pl.ds(base + s * r5, r5)])
                    pltpu.sync_copy(acc_c.at[pl.ds(s * r5, r5)],
                                    cnt_hbm.at[pl.ds(base + s * r5, r5)])
            plsc.subcore_barrier()


def _run_rv_acc(vfA, vfB, logits, src, dst, gmax16, R):
    E = src.shape[0]
    acc_rows = R // 4 + 512
    acc_rows += (-acc_rows) % (_NS * 800)
    mesh = plsc.VectorSubcoreMesh(core_axis_name="c", subcore_axis_name="s")
    f = functools.partial(
        pl.kernel,
        out_type=[jax.ShapeDtypeStruct((R, 32), jnp.float32),
                  jax.ShapeDtypeStruct((R, 32), jnp.float32),
                  jax.ShapeDtypeStruct((R,), jnp.float32),
                  jax.ShapeDtypeStruct((R,), jnp.float32)],
        mesh=mesh,
        scratch_types=[
            pltpu.VMEM((_NSL, 128), jnp.int32),
            pltpu.VMEM((_NSL, 128), jnp.int32),
            pltpu.VMEM((_NSL, 128), jnp.int32),
            pltpu.VMEM((_W,), jnp.float32),
            pltpu.VMEM((_W,), jnp.float32),
            pltpu.VMEM((_W, 32), jnp.float32),
            pltpu.VMEM((_W,), jnp.float32),
            pltpu.VMEM((16,), jnp.float32),
            pltpu.VMEM((800, 32), jnp.float32),
            pltpu.VMEM((acc_rows // _NS,), jnp.float32),
            pltpu.VMEM_SHARED((acc_rows, 32), jnp.float32),
            pltpu.VMEM_SHARED((acc_rows,), jnp.float32),
            pltpu.VMEM_SHARED((acc_rows,), jnp.float32),
            pltpu.SemaphoreType.DMA,
            pltpu.SemaphoreType.DMA,
        ],
        compiler_params=pltpu.CompilerParams(use_tc_tiling_on_sc=False),
    )(_rv_acc_body)
    return f(vfA, vfB, logits, src, dst, gmax16)


def _fill(ref, vec):
    if len(ref.shape) == 1:
        def _f(i, _):
            ref[pl.ds(i * _L, _L)] = vec
            return 0
        lax.fori_loop(0, ref.shape[0] // _L, _f, 0)
    else:
        hs = ref.shape[1] // _L

        def _f(i, _):
            ref[i // hs, pl.ds((i % hs) * _L, _L)] = vec
            return 0
        lax.fori_loop(0, ref.shape[0] * hs, _f, 0)


def _rr_body(A_hbm, B_hbm, src_hbm, dst_hbm, attr_hbm, wb_hbm,
             esum_hbm, ecnt_hbm,
             idx_s, idx_d, loc, attr_v, a_buf, b_buf, ones_v, wb_v,
             zero_buf, zero1_v, acc_sh, cnt_sh, sem_a, sem_b, sem_w):
    c = lax.axis_index("c")
    s = lax.axis_index("s")
    R = A_hbm.shape[0]
    Rq = R // 4
    ACC = acc_sh.shape[0]
    E = src_hbm.shape[0]

    _fill(zero_buf, jnp.zeros((_L,), jnp.float32))
    _fill(zero1_v, jnp.zeros((_L,), jnp.float32))
    _fill(ones_v, jnp.ones((_L,), jnp.float32))
    pltpu.sync_copy(wb_hbm, wb_v)
    wv = [wb_v[0, pl.ds(h * _L, _L)] for h in range(2)]
    bb = [wb_v[1, pl.ds(h * _L, _L)] for h in range(2)]

    rows_per_tile = ACC // _NS
    n_chunks = (E // _W)
    my_n = (n_chunks + _NS - 1 - s) // _NS
    iota = lax.iota(jnp.int32, _L)

    for q in range(2):
        base = (c * 2 + q) * Rq

        for z in range(rows_per_tile // zero_buf.shape[0]):
            pltpu.sync_copy(
                zero_buf,
                acc_sh.at[pl.ds(s * rows_per_tile + z * zero_buf.shape[0],
                                zero_buf.shape[0])])
        pltpu.sync_copy(zero1_v.at[pl.ds(0, rows_per_tile)],
                        cnt_sh.at[pl.ds(s * rows_per_tile, rows_per_tile)])
        plsc.subcore_barrier()

        def _chunk(k, _):
            g = s + _NS * k
            e0 = g * _W
            for j in range(_NSL):
                pltpu.sync_copy(src_hbm.at[pl.ds(e0 + j * 128, 128)],
                                idx_s.at[j])
                pltpu.sync_copy(dst_hbm.at[pl.ds(e0 + j * 128, 128)],
                                idx_d.at[j])
            pltpu.sync_copy(attr_hbm.at[pl.ds(e0, _W)], attr_v)
            cp_a = [pltpu.make_async_copy(
                A_hbm.at[idx_s.at[j]], a_buf.at[pl.ds(j * 128, 128)], sem_a)
                for j in range(_NSL)]
            cp_b = [pltpu.make_async_copy(
                B_hbm.at[idx_d.at[j]], b_buf.at[pl.ds(j * 128, 128)], sem_b)
                for j in range(_NSL)]
            for cp in cp_a + cp_b:
                cp.start()
            for cp in cp_a + cp_b:
                cp.wait()

            def _locs(i, _):
                sv = idx_s[i // 8, pl.ds((i % 8) * _L, _L)]
                lo = sv - base
                inb = (lo >= 0) & (lo < Rq)
                trash = Rq + ((i * _L + iota) & 511)
                safe = jnp.where(inb, lo, trash)
                loc[i // 8, pl.ds((i % 8) * _L, _L)] = safe
                return 0
            lax.fori_loop(0, _W // _L, _locs, 0)

            def _edge(gi, _):
                av16 = attr_v[pl.ds(gi * _L, _L)]
                for j2 in range(_L):
                    j = gi * _L + j2
                    av = av16[j2]
                    for h in range(2):
                        va = a_buf[j, pl.ds(h * _L, _L)]
                        vb = b_buf[j, pl.ds(h * _L, _L)]
                        a_buf[j, pl.ds(h * _L, _L)] = _stanh(
                            va + vb + av * wv[h] + bb[h])
                return 0
            lax.fori_loop(0, _W // _L, _edge, 0)

            for j in range(_NSL):
                pltpu.sync_copy(a_buf.at[pl.ds(j * 128, 128)],
                                acc_sh.at[loc.at[j]], add=True)
                pltpu.sync_copy(ones_v.at[pl.ds(j * 128, 128)],
                                cnt_sh.at[loc.at[j]], add=True)
            return 0

        lax.fori_loop(0, my_n, _chunk, 0)
        plsc.subcore_barrier()

        r10 = Rq // 10
        r5 = Rq // 5

        @pl.when(s < 10)
        def _():
            pltpu.sync_copy(acc_sh.at[pl.ds(s * r10, r10)],
                            esum_hbm.at[pl.ds(base + s * r10, r10)])

        @pl.when(s < 5)
        def _():
            pltpu.sync_copy(cnt_sh.at[pl.ds(s * r5, r5)],
                            ecnt_hbm.at[pl.ds(base + s * r5, r5)])
        plsc.subcore_barrier()


def _run_rr(A, B, src, dst, attr, wb):
    R = A.shape[0]
    acc_rows = R // 4 + 512
    acc_rows += (-acc_rows) % (_NS * 800)
    mesh = plsc.VectorSubcoreMesh(core_axis_name="c", subcore_axis_name="s")
    f = functools.partial(
        pl.kernel,
        out_type=[jax.ShapeDtypeStruct((R, 32), jnp.float32),
                  jax.ShapeDtypeStruct((R,), jnp.float32)],
        mesh=mesh,
        scratch_types=[
            pltpu.VMEM((_NSL, 128), jnp.int32),
            pltpu.VMEM((_NSL, 128), jnp.int32),
            pltpu.VMEM((_NSL, 128), jnp.int32),
            pltpu.VMEM((_W,), jnp.float32),
            pltpu.VMEM((_W, 32), jnp.float32),
            pltpu.VMEM((_W, 32), jnp.float32),
            pltpu.VMEM((_W,), jnp.float32),
            pltpu.VMEM((2, 32), jnp.float32),
            pltpu.VMEM((800, 32), jnp.float32),
            pltpu.VMEM((acc_rows // _NS, ), jnp.float32),
            pltpu.VMEM_SHARED((acc_rows, 32), jnp.float32),
            pltpu.VMEM_SHARED((acc_rows,), jnp.float32),
            pltpu.SemaphoreType.DMA,
            pltpu.SemaphoreType.DMA,
            pltpu.SemaphoreType.DMA,
        ],
        compiler_params=pltpu.CompilerParams(use_tc_tiling_on_sc=False),
    )(_rr_body)
    return f(A, B, src, dst, attr, wb)


_BR = 2000


def _head_body(reqx_ref, esum_ref, ecnt_ref, nsum_ref, d_ref, cnt_ref,
               wr_ref, br_ref, wn1_ref, wn2_ref, bn_ref,
               w1_ref, b1_ref, w2_ref, b2_ref, w3_ref, b3_ref, out_ref):
    reqx = reqx_ref[...]
    req_feat = jnp.dot(reqx, wr_ref[...],
                       preferred_element_type=jnp.float32) + br_ref[...]
    edge_mean = esum_ref[...] / jnp.maximum(ecnt_ref[...], 1.0)
    trip = (jnp.dot(reqx, wn1_ref[...], preferred_element_type=jnp.float32)
            + jnp.dot(edge_mean, wn2_ref[...],
                      preferred_element_type=jnp.float32) + bn_ref[...])
    veh_agg = nsum_ref[...] / (jnp.maximum(d_ref[...], 1e-30)
                               * jnp.maximum(cnt_ref[...], 1.0))
    act = jnp.concatenate([req_feat, trip, veh_agg], axis=1)
    h = jnp.tanh(jnp.dot(act, w1_ref[...],
                         preferred_element_type=jnp.float32) + b1_ref[...])
    h = jnp.tanh(jnp.dot(h, w2_ref[...],
                         preferred_element_type=jnp.float32) + b2_ref[...])
    out_ref[...] = jnp.dot(h, w3_ref[...],
                           preferred_element_type=jnp.float32) + b3_ref[...]


def _run_head(reqx, esum, ecnt, nsum, d, cnt,
              W_req, b_req, Wn1, Wn2, bn, W1, b1, W2, b2, W3, b3):
    R = reqx.shape[0]
    grid = (R // _BR,)
    row = lambda i: (i, 0)
    full = lambda i: (0, 0)
    return pl.pallas_call(
        _head_body,
        grid=grid,
        in_specs=[
            pl.BlockSpec((_BR, 8), row),
            pl.BlockSpec((_BR, 32), row),
            pl.BlockSpec((_BR, 1), row),
            pl.BlockSpec((_BR, 64), row),
            pl.BlockSpec((_BR, 1), row),
            pl.BlockSpec((_BR, 1), row),
            pl.BlockSpec((8, 32), full),
            pl.BlockSpec((1, 32), full),
            pl.BlockSpec((8, 32), full),
            pl.BlockSpec((32, 32), full),
            pl.BlockSpec((1, 32), full),
            pl.BlockSpec((128, 512), full),
            pl.BlockSpec((1, 512), full),
            pl.BlockSpec((512, 512), full),
            pl.BlockSpec((1, 512), full),
            pl.BlockSpec((512, 1), full),
            pl.BlockSpec((1, 1), full),
        ],
        out_specs=pl.BlockSpec((_BR, 1), row),
        out_shape=jax.ShapeDtypeStruct((R, 1), jnp.float32),
    )(reqx, esum, ecnt, nsum, d, cnt,
      W_req, b_req.reshape(1, -1), Wn1, Wn2, bn.reshape(1, -1),
      W1, b1.reshape(1, -1), W2, b2.reshape(1, -1), W3, b3.reshape(1, -1))


def kernel(passengers_x, vehicles_x, requests_x,
           veh2pas_sender_edge_index, veh2pas_receiver_edge_index,
           req2req_edge_index, req2req_edge_attr,
           req2veh_sender_edge_index, req2veh_receiver_edge_index,
           req2veh_edge_attr,
           W_pas, b_pas, W_veh, b_veh, W_req, b_req,
           W_trip_e, b_trip_e, W_trip_n, b_trip_n, W_rv, b_rv,
           W_att1, b_att1, W_att2, b_att2,
           W_act1, b_act1, W_act2, b_act2, W_act3, b_act3):
    R = requests_x.shape[0]
    V = vehicles_x.shape[0]

    pas_feat = passengers_x @ W_pas + b_pas
    g = pas_feat[veh2pas_receiver_edge_index]
    vp_sum = jax.ops.segment_sum(g, veh2pas_sender_edge_index, num_segments=V)
    vp_cnt = jax.ops.segment_sum(
        jnp.ones((g.shape[0], 1), jnp.float32),
        veh2pas_sender_edge_index, num_segments=V)
    pas_mean = vp_sum / jnp.maximum(vp_cnt, 1.0)
    veh_feat = jnp.concatenate(
        [vehicles_x @ W_veh + b_veh, pas_mean], axis=1)

    A = requests_x @ W_trip_e[:8]
    B = requests_x @ W_trip_e[8:16]
    w_attr = W_trip_e[16]
    src_rr = req2req_edge_index[0]
    dst_rr = req2req_edge_index[1]
    wb = jnp.stack([w_attr, b_trip_e])
    esum, ecnt1 = _run_rr(A, B, src_rr, dst_rr,
                          req2req_edge_attr.reshape(-1), wb)
    ecnt = ecnt1.reshape(-1, 1)

    src_rv = req2veh_sender_edge_index
    dst_rv = req2veh_receiver_edge_index
    Creq = requests_x @ (W_rv[:8] @ W_att1[64:96])
    Cveh = (veh_feat @ W_att1[:64]
            + (vehicles_x @ W_rv[8:13] + b_rv) @ W_att1[64:96]
            + b_att1)
    U = W_rv[13:15] @ W_att1[64:96]
    cst = jnp.zeros((4, 96), jnp.float32)
    cst = cst.at[0].set(U[0]).at[1].set(U[1])
    cst = cst.at[2].set(W_att2[:, 0]).at[3, 0].set(b_att2[0])
    logits = _run_rv_logits(Creq, Cveh, src_rv, dst_rv,
                            req2veh_edge_attr[:, 0], req2veh_edge_attr[:, 1],
                            cst)
    gmax = jnp.max(logits)
    gmax16 = jnp.full((16,), gmax, jnp.float32)
    vfA = veh_feat[:, :32]
    vfB = veh_feat[:, 32:]
    nsumA, nsumB, dsum1, rvcnt1 = _run_rv_acc(
        vfA, vfB, logits, src_rv, dst_rv, gmax16, R)
    nsum = jnp.concatenate([nsumA, nsumB], axis=1)
    dsum = dsum1.reshape(-1, 1)
    rvcnt = rvcnt1.reshape(-1, 1)

    Wn1 = W_trip_n[:8]
    Wn2 = W_trip_n[8:40]
    return _run_head(requests_x, esum, ecnt, nsum, dsum, rvcnt,
                     W_req, b_req, Wn1, Wn2, b_trip_n,
                     W_act1, b_act1, W_act2, b_act2, W_act3, b_act3)


# speedup vs baseline: 1.3541x; 1.0128x over previous
import functools

import jax
import jax.numpy as jnp
from jax import lax
from jax.experimental import pallas as pl
from jax.experimental.pallas import tpu as pltpu
from jax.experimental.pallas import tpu_sc as plsc

_NC, _NS, _L = 2, 16, 16
_W = 512
_NSL = _W // 128


def _stanh(x):
    return 1.0 - 2.0 / (jnp.exp(2.0 * x) + 1.0)


def _rv_logit_body(creq_hbm, cveh_hbm, src_hbm, dst_hbm, a0_hbm, a1_hbm,
                   cst_hbm, logit_hbm,
                   idx_s, idx_d, a0_v, a1_v, cr_buf, cv_buf, out_v, cst_v,
                   sems):
    c = lax.axis_index("c")
    s = lax.axis_index("s")
    wid = c * _NS + s
    NW = _NC * _NS
    W2 = 256
    NS2 = W2 // 128
    E = src_hbm.shape[0]
    n_chunks = E // W2
    my_n = (n_chunks + NW - 1 - wid) // NW

    pltpu.sync_copy(cst_hbm, cst_v)
    u0 = [cst_v[0, pl.ds(h * _L, _L)] for h in range(6)]
    u1 = [cst_v[1, pl.ds(h * _L, _L)] for h in range(6)]
    w2 = [cst_v[2, pl.ds(h * _L, _L)] for h in range(6)]
    batt = cst_v[3, pl.ds(0, _L)][0]
    iota = lax.iota(jnp.int32, _L)

    def _cps(b):
        return ([pltpu.make_async_copy(
                     creq_hbm.at[idx_s[b].at[j]],
                     cr_buf[b].at[pl.ds(j * 128, 128)], sems[b])
                 for j in range(NS2)]
                + [pltpu.make_async_copy(
                     cveh_hbm.at[idx_d[b].at[j]],
                     cv_buf[b].at[pl.ds(j * 128, 128)], sems[b])
                   for j in range(NS2)])

    def _stage(k, b):
        @pl.when(k < my_n)
        def _():
            e0 = (wid + NW * k) * W2
            for j in range(NS2):
                pltpu.sync_copy(src_hbm.at[pl.ds(e0 + j * 128, 128)],
                                idx_s[b].at[j])
                pltpu.sync_copy(dst_hbm.at[pl.ds(e0 + j * 128, 128)],
                                idx_d[b].at[j])
            pltpu.sync_copy(a0_hbm.at[pl.ds(e0, W2)], a0_v[b])
            pltpu.sync_copy(a1_hbm.at[pl.ds(e0, W2)], a1_v[b])
            for cp in _cps(b):
                cp.start()

    def _compute(k, b):
        @pl.when(k < my_n)
        def _():
            for cp in _cps(b):
                cp.wait()

            def _grp(gi, _):
                av0 = a0_v[b][pl.ds(gi * _L, _L)]
                av1 = a1_v[b][pl.ds(gi * _L, _L)]
                res = jnp.zeros((_L,), jnp.float32)
                for j2 in range(_L):
                    j = gi * _L + j2
                    s0 = av0[j2]
                    s1 = av1[j2]
                    acc = jnp.zeros((_L,), jnp.float32)
                    for h in range(6):
                        z = (cr_buf[b][j, pl.ds(h * _L, _L)]
                             + cv_buf[b][j, pl.ds(h * _L, _L)]
                             + s0 * u0[h] + s1 * u1[h])
                        acc = acc + _stanh(z) * w2[h]
                    lg = batt
                    for kl in range(_L):
                        lg = lg + acc[kl]
                    res = jnp.where(iota == j2, jnp.full((_L,), lg), res)
                out_v[b][pl.ds(gi * _L, _L)] = res
                return 0
            lax.fori_loop(0, W2 // _L, _grp, 0)
            e0 = (wid + NW * k) * W2
            pltpu.sync_copy(out_v[b], logit_hbm.at[pl.ds(e0, W2)])

    _stage(0, 0)

    def _pair(p, _):
        k0 = 2 * p
        _stage(k0 + 1, 1)
        _compute(k0, 0)
        _stage(k0 + 2, 0)
        _compute(k0 + 1, 1)
        return 0
    lax.fori_loop(0, (my_n + 1) // 2, _pair, 0)


def _run_rv_logits(Creq, Cveh, src, dst, a0, a1, cst):
    E = src.shape[0]
    mesh = plsc.VectorSubcoreMesh(core_axis_name="c", subcore_axis_name="s")
    f = functools.partial(
        pl.kernel,
        out_type=jax.ShapeDtypeStruct((E,), jnp.float32),
        mesh=mesh,
        scratch_types=[
            [pltpu.VMEM((2, 128), jnp.int32)] * 2,
            [pltpu.VMEM((2, 128), jnp.int32)] * 2,
            [pltpu.VMEM((256,), jnp.float32)] * 2,
            [pltpu.VMEM((256,), jnp.float32)] * 2,
            [pltpu.VMEM((256, 96), jnp.float32)] * 2,
            [pltpu.VMEM((256, 96), jnp.float32)] * 2,
            [pltpu.VMEM((256,), jnp.float32)] * 2,
            pltpu.VMEM((4, 96), jnp.float32),
            [pltpu.SemaphoreType.DMA] * 2,
        ],
        compiler_params=pltpu.CompilerParams(use_tc_tiling_on_sc=False),
    )(_rv_logit_body)
    return f(Creq, Cveh, src, dst, a0, a1, cst)


def _rv_acc_body(vfA_hbm, vfB_hbm, lg_hbm, src_hbm, dst_hbm, gmax_hbm,
                 nsumA_hbm, nsumB_hbm, d_hbm, cnt_hbm,
                 idx_s, idx_d, loc, lg_v, e_v, vf_buf, ones_v, gmax_v,
                 zero_buf, zero1_v, acc_n, acc_d, acc_c, sem_a, sem_w):
    c = lax.axis_index("c")
    s = lax.axis_index("s")
    R = d_hbm.shape[0]
    Rq = R // 4
    ACC = acc_n.shape[0]
    E = src_hbm.shape[0]

    _fill(zero_buf, jnp.zeros((_L,), jnp.float32))
    _fill(zero1_v, jnp.zeros((_L,), jnp.float32))
    _fill(ones_v, jnp.ones((_L,), jnp.float32))
    pltpu.sync_copy(gmax_hbm, gmax_v)
    gmax = gmax_v[pl.ds(0, _L)][0]

    rows_per_tile = ACC // _NS
    n_chunks = E // _W
    my_n = (n_chunks + _NS - 1 - s) // _NS
    iota = lax.iota(jnp.int32, _L)

    for q in range(2):
        base = (c * 2 + q) * Rq
        for fh in range(2):
            vf_hbm = vfA_hbm if fh == 0 else vfB_hbm
            nout_hbm = nsumA_hbm if fh == 0 else nsumB_hbm

            for z in range(rows_per_tile // zero_buf.shape[0]):
                pltpu.sync_copy(
                    zero_buf,
                    acc_n.at[pl.ds(s * rows_per_tile + z * zero_buf.shape[0],
                                   zero_buf.shape[0])])
            if fh == 0:
                pltpu.sync_copy(
                    zero1_v.at[pl.ds(0, rows_per_tile)],
                    acc_d.at[pl.ds(s * rows_per_tile, rows_per_tile)])
                pltpu.sync_copy(
                    zero1_v.at[pl.ds(0, rows_per_tile)],
                    acc_c.at[pl.ds(s * rows_per_tile, rows_per_tile)])
            plsc.subcore_barrier()

            def _chunk(k, _):
                g = s + _NS * k
                e0 = g * _W
                for j in range(_NSL):
                    pltpu.sync_copy(src_hbm.at[pl.ds(e0 + j * 128, 128)],
                                    idx_s.at[j])
                    pltpu.sync_copy(dst_hbm.at[pl.ds(e0 + j * 128, 128)],
                                    idx_d.at[j])
                pltpu.sync_copy(lg_hbm.at[pl.ds(e0, _W)], lg_v)
                cps = [pltpu.make_async_copy(
                    vf_hbm.at[idx_d.at[j]],
                    vf_buf.at[pl.ds(j * 128, 128)], sem_a)
                    for j in range(_NSL)]
                for cp in cps:
                    cp.start()

                def _locs(i, _):
                    sv = idx_s[i // 8, pl.ds((i % 8) * _L, _L)]
                    lo = sv - base
                    inb = (lo >= 0) & (lo < Rq)
                    trash = Rq + ((i * _L + iota) & 511)
                    loc[i // 8, pl.ds((i % 8) * _L, _L)] = (
                        jnp.where(inb, lo, trash))
                    ev = jnp.exp(lg_v[pl.ds(i * _L, _L)] - gmax)
                    e_v[pl.ds(i * _L, _L)] = ev
                    return 0
                lax.fori_loop(0, _W // _L, _locs, 0)
                for cp in cps:
                    cp.wait()

                def _rows(gi, _):
                    ev16 = e_v[pl.ds(gi * _L, _L)]
                    for j2 in range(_L):
                        j = gi * _L + j2
                        e = ev16[j2]
                        for h in range(2):
                            vf_buf[j, pl.ds(h * _L, _L)] = (
                                vf_buf[j, pl.ds(h * _L, _L)] * e)
                    return 0
                lax.fori_loop(0, _W // _L, _rows, 0)

                for j in range(_NSL):
                    pltpu.sync_copy(vf_buf.at[pl.ds(j * 128, 128)],
                                    acc_n.at[loc.at[j]], add=True)
                    if fh == 0:
                        pltpu.sync_copy(e_v.at[pl.ds(j * 128, 128)],
                                        acc_d.at[loc.at[j]], add=True)
                        pltpu.sync_copy(ones_v.at[pl.ds(j * 128, 128)],
                                        acc_c.at[loc.at[j]], add=True)
                return 0

            lax.fori_loop(0, my_n, _chunk, 0)
            plsc.subcore_barrier()

            r10 = Rq // 10
            r5 = Rq // 5

            @pl.when(s < 10)
            def _():
                pltpu.sync_copy(acc_n.at[pl.ds(s * r10, r10)],
                                nout_hbm.at[pl.ds(base + s * r10, r10)])

            if fh == 0:
                @pl.when(s < 5)
                def _():
                    pltpu.sync_copy(acc_d.at[pl.ds(s * r5, r5)],
                                    d_hbm.at[pl.ds(base + s * r5, r5)])
                    pltpu.sync_copy(acc_c.at[pl.ds(s * r5, r5)],
                                    cnt_hbm.at[pl.ds(base + s * r5, r5)])
            plsc.subcore_barrier()


def _run_rv_acc(vfA, vfB, logits, src, dst, gmax16, R):
    E = src.shape[0]
    acc_rows = R // 4 + 512
    acc_rows += (-acc_rows) % (_NS * 800)
    mesh = plsc.VectorSubcoreMesh(core_axis_name="c", subcore_axis_name="s")
    f = functools.partial(
        pl.kernel,
        out_type=[jax.ShapeDtypeStruct((R, 32), jnp.float32),
                  jax.ShapeDtypeStruct((R, 32), jnp.float32),
                  jax.ShapeDtypeStruct((R,), jnp.float32),
                  jax.ShapeDtypeStruct((R,), jnp.float32)],
        mesh=mesh,
        scratch_types=[
            pltpu.VMEM((_NSL, 128), jnp.int32),
            pltpu.VMEM((_NSL, 128), jnp.int32),
            pltpu.VMEM((_NSL, 128), jnp.int32),
            pltpu.VMEM((_W,), jnp.float32),
            pltpu.VMEM((_W,), jnp.float32),
            pltpu.VMEM((_W, 32), jnp.float32),
            pltpu.VMEM((_W,), jnp.float32),
            pltpu.VMEM((16,), jnp.float32),
            pltpu.VMEM((800, 32), jnp.float32),
            pltpu.VMEM((acc_rows // _NS,), jnp.float32),
            pltpu.VMEM_SHARED((acc_rows, 32), jnp.float32),
            pltpu.VMEM_SHARED((acc_rows,), jnp.float32),
            pltpu.VMEM_SHARED((acc_rows,), jnp.float32),
            pltpu.SemaphoreType.DMA,
            pltpu.SemaphoreType.DMA,
        ],
        compiler_params=pltpu.CompilerParams(use_tc_tiling_on_sc=False),
    )(_rv_acc_body)
    return f(vfA, vfB, logits, src, dst, gmax16)


def _fill(ref, vec):
    if len(ref.shape) == 1:
        def _f(i, _):
            ref[pl.ds(i * _L, _L)] = vec
            return 0
        lax.fori_loop(0, ref.shape[0] // _L, _f, 0)
    else:
        hs = ref.shape[1] // _L

        def _f(i, _):
            ref[i // hs, pl.ds((i % hs) * _L, _L)] = vec
            return 0
        lax.fori_loop(0, ref.shape[0] * hs, _f, 0)


def _rr_body(A_hbm, B_hbm, src_hbm, dst_hbm, attr_hbm, wb_hbm,
             esum_hbm, ecnt_hbm,
             idx_s, idx_d, loc, attr_v, a_buf, b_buf, ones_v, wb_v,
             zero_buf, zero1_v, acc_sh, cnt_sh, sem_a, sem_b, sem_w):
    c = lax.axis_index("c")
    s = lax.axis_index("s")
    R = A_hbm.shape[0]
    Rq = R // 4
    ACC = acc_sh.shape[0]
    E = src_hbm.shape[0]

    _fill(zero_buf, jnp.zeros((_L,), jnp.float32))
    _fill(zero1_v, jnp.zeros((_L,), jnp.float32))
    _fill(ones_v, jnp.ones((_L,), jnp.float32))
    pltpu.sync_copy(wb_hbm, wb_v)
    wv = [wb_v[0, pl.ds(h * _L, _L)] for h in range(2)]
    bb = [wb_v[1, pl.ds(h * _L, _L)] for h in range(2)]

    rows_per_tile = ACC // _NS
    n_chunks = (E // _W)
    my_n = (n_chunks + _NS - 1 - s) // _NS
    iota = lax.iota(jnp.int32, _L)

    for q in range(2):
        base = (c * 2 + q) * Rq

        for z in range(rows_per_tile // zero_buf.shape[0]):
            pltpu.sync_copy(
                zero_buf,
                acc_sh.at[pl.ds(s * rows_per_tile + z * zero_buf.shape[0],
                                zero_buf.shape[0])])
        pltpu.sync_copy(zero1_v.at[pl.ds(0, rows_per_tile)],
                        cnt_sh.at[pl.ds(s * rows_per_tile, rows_per_tile)])
        plsc.subcore_barrier()

        def _chunk(k, _):
            g = s + _NS * k
            e0 = g * _W
            for j in range(_NSL):
                pltpu.sync_copy(src_hbm.at[pl.ds(e0 + j * 128, 128)],
                                idx_s.at[j])
                pltpu.sync_copy(dst_hbm.at[pl.ds(e0 + j * 128, 128)],
                                idx_d.at[j])
            pltpu.sync_copy(attr_hbm.at[pl.ds(e0, _W)], attr_v)
            cp_a = [pltpu.make_async_copy(
                A_hbm.at[idx_s.at[j]], a_buf.at[pl.ds(j * 128, 128)], sem_a)
                for j in range(_NSL)]
            cp_b = [pltpu.make_async_copy(
                B_hbm.at[idx_d.at[j]], b_buf.at[pl.ds(j * 128, 128)], sem_b)
                for j in range(_NSL)]
            for cp in cp_a + cp_b:
                cp.start()
            for cp in cp_a + cp_b:
                cp.wait()

            def _locs(i, _):
                sv = idx_s[i // 8, pl.ds((i % 8) * _L, _L)]
                lo = sv - base
                inb = (lo >= 0) & (lo < Rq)
                trash = Rq + ((i * _L + iota) & 511)
                safe = jnp.where(inb, lo, trash)
                loc[i // 8, pl.ds((i % 8) * _L, _L)] = safe
                return 0
            lax.fori_loop(0, _W // _L, _locs, 0)

            def _edge(gi, _):
                av16 = attr_v[pl.ds(gi * _L, _L)]
                for j2 in range(_L):
                    j = gi * _L + j2
                    av = av16[j2]
                    for h in range(2):
                        va = a_buf[j, pl.ds(h * _L, _L)]
                        vb = b_buf[j, pl.ds(h * _L, _L)]
                        a_buf[j, pl.ds(h * _L, _L)] = _stanh(
                            va + vb + av * wv[h] + bb[h])
                return 0
            lax.fori_loop(0, _W // _L, _edge, 0)

            for j in range(_NSL):
                pltpu.sync_copy(a_buf.at[pl.ds(j * 128, 128)],
                                acc_sh.at[loc.at[j]], add=True)
                pltpu.sync_copy(ones_v.at[pl.ds(j * 128, 128)],
                                cnt_sh.at[loc.at[j]], add=True)
            return 0

        lax.fori_loop(0, my_n, _chunk, 0)
        plsc.subcore_barrier()

        r10 = Rq // 10
        r5 = Rq // 5

        @pl.when(s < 10)
        def _():
            pltpu.sync_copy(acc_sh.at[pl.ds(s * r10, r10)],
                            esum_hbm.at[pl.ds(base + s * r10, r10)])

        @pl.when(s < 5)
        def _():
            pltpu.sync_copy(cnt_sh.at[pl.ds(s * r5, r5)],
                            ecnt_hbm.at[pl.ds(base + s * r5, r5)])
        plsc.subcore_barrier()


def _run_rr(A, B, src, dst, attr, wb):
    R = A.shape[0]
    acc_rows = R // 4 + 512
    acc_rows += (-acc_rows) % (_NS * 800)
    mesh = plsc.VectorSubcoreMesh(core_axis_name="c", subcore_axis_name="s")
    f = functools.partial(
        pl.kernel,
        out_type=[jax.ShapeDtypeStruct((R, 32), jnp.float32),
                  jax.ShapeDtypeStruct((R,), jnp.float32)],
        mesh=mesh,
        scratch_types=[
            pltpu.VMEM((_NSL, 128), jnp.int32),
            pltpu.VMEM((_NSL, 128), jnp.int32),
            pltpu.VMEM((_NSL, 128), jnp.int32),
            pltpu.VMEM((_W,), jnp.float32),
            pltpu.VMEM((_W, 32), jnp.float32),
            pltpu.VMEM((_W, 32), jnp.float32),
            pltpu.VMEM((_W,), jnp.float32),
            pltpu.VMEM((2, 32), jnp.float32),
            pltpu.VMEM((800, 32), jnp.float32),
            pltpu.VMEM((acc_rows // _NS, ), jnp.float32),
            pltpu.VMEM_SHARED((acc_rows, 32), jnp.float32),
            pltpu.VMEM_SHARED((acc_rows,), jnp.float32),
            pltpu.SemaphoreType.DMA,
            pltpu.SemaphoreType.DMA,
            pltpu.SemaphoreType.DMA,
        ],
        compiler_params=pltpu.CompilerParams(use_tc_tiling_on_sc=False),
    )(_rr_body)
    return f(A, B, src, dst, attr, wb)


_BR = 2000


def _head_body(reqx_ref, esum_ref, ecnt_ref, nsum_ref, d_ref, cnt_ref,
               wr_ref, br_ref, wn1_ref, wn2_ref, bn_ref,
               w1_ref, b1_ref, w2_ref, b2_ref, w3_ref, b3_ref, out_ref):
    reqx = reqx_ref[...]
    req_feat = jnp.dot(reqx, wr_ref[...],
                       preferred_element_type=jnp.float32) + br_ref[...]
    edge_mean = esum_ref[...] / jnp.maximum(ecnt_ref[...], 1.0)
    trip = (jnp.dot(reqx, wn1_ref[...], preferred_element_type=jnp.float32)
            + jnp.dot(edge_mean, wn2_ref[...],
                      preferred_element_type=jnp.float32) + bn_ref[...])
    veh_agg = nsum_ref[...] / (jnp.maximum(d_ref[...], 1e-30)
                               * jnp.maximum(cnt_ref[...], 1.0))
    act = jnp.concatenate([req_feat, trip, veh_agg], axis=1)
    h = jnp.tanh(jnp.dot(act, w1_ref[...],
                         preferred_element_type=jnp.float32) + b1_ref[...])
    h = jnp.tanh(jnp.dot(h, w2_ref[...],
                         preferred_element_type=jnp.float32) + b2_ref[...])
    out_ref[...] = jnp.dot(h, w3_ref[...],
                           preferred_element_type=jnp.float32) + b3_ref[...]


def _run_head(reqx, esum, ecnt, nsum, d, cnt,
              W_req, b_req, Wn1, Wn2, bn, W1, b1, W2, b2, W3, b3):
    R = reqx.shape[0]
    grid = (R // _BR,)
    row = lambda i: (i, 0)
    full = lambda i: (0, 0)
    return pl.pallas_call(
        _head_body,
        grid=grid,
        in_specs=[
            pl.BlockSpec((_BR, 8), row),
            pl.BlockSpec((_BR, 32), row),
            pl.BlockSpec((_BR, 1), row),
            pl.BlockSpec((_BR, 64), row),
            pl.BlockSpec((_BR, 1), row),
            pl.BlockSpec((_BR, 1), row),
            pl.BlockSpec((8, 32), full),
            pl.BlockSpec((1, 32), full),
            pl.BlockSpec((8, 32), full),
            pl.BlockSpec((32, 32), full),
            pl.BlockSpec((1, 32), full),
            pl.BlockSpec((128, 512), full),
            pl.BlockSpec((1, 512), full),
            pl.BlockSpec((512, 512), full),
            pl.BlockSpec((1, 512), full),
            pl.BlockSpec((512, 1), full),
            pl.BlockSpec((1, 1), full),
        ],
        out_specs=pl.BlockSpec((_BR, 1), row),
        out_shape=jax.ShapeDtypeStruct((R, 1), jnp.float32),
    )(reqx, esum, ecnt, nsum, d, cnt,
      W_req, b_req.reshape(1, -1), Wn1, Wn2, bn.reshape(1, -1),
      W1, b1.reshape(1, -1), W2, b2.reshape(1, -1), W3, b3.reshape(1, -1))


def kernel(passengers_x, vehicles_x, requests_x,
           veh2pas_sender_edge_index, veh2pas_receiver_edge_index,
           req2req_edge_index, req2req_edge_attr,
           req2veh_sender_edge_index, req2veh_receiver_edge_index,
           req2veh_edge_attr,
           W_pas, b_pas, W_veh, b_veh, W_req, b_req,
           W_trip_e, b_trip_e, W_trip_n, b_trip_n, W_rv, b_rv,
           W_att1, b_att1, W_att2, b_att2,
           W_act1, b_act1, W_act2, b_act2, W_act3, b_act3):
    R = requests_x.shape[0]
    V = vehicles_x.shape[0]

    pas_feat = passengers_x @ W_pas + b_pas
    g = pas_feat[veh2pas_receiver_edge_index]
    vp_sum = jax.ops.segment_sum(g, veh2pas_sender_edge_index, num_segments=V)
    vp_cnt = jax.ops.segment_sum(
        jnp.ones((g.shape[0], 1), jnp.float32),
        veh2pas_sender_edge_index, num_segments=V)
    pas_mean = vp_sum / jnp.maximum(vp_cnt, 1.0)
    veh_feat = jnp.concatenate(
        [vehicles_x @ W_veh + b_veh, pas_mean], axis=1)

    A = requests_x @ W_trip_e[:8]
    B = requests_x @ W_trip_e[8:16]
    w_attr = W_trip_e[16]
    src_rr = req2req_edge_index[0]
    dst_rr = req2req_edge_index[1]
    wb = jnp.stack([w_attr, b_trip_e])
    esum, ecnt1 = _run_rr(A, B, src_rr, dst_rr,
                          req2req_edge_attr.reshape(-1), wb)
    ecnt = ecnt1.reshape(-1, 1)

    src_rv = req2veh_sender_edge_index
    dst_rv = req2veh_receiver_edge_index
    Creq = requests_x @ (W_rv[:8] @ W_att1[64:96])
    Cveh = (veh_feat @ W_att1[:64]
            + (vehicles_x @ W_rv[8:13] + b_rv) @ W_att1[64:96]
            + b_att1)
    U = W_rv[13:15] @ W_att1[64:96]
    cst = jnp.zeros((4, 96), jnp.float32)
    cst = cst.at[0].set(U[0]).at[1].set(U[1])
    cst = cst.at[2].set(W_att2[:, 0]).at[3, 0].set(b_att2[0])
    logits = _run_rv_logits(Creq, Cveh, src_rv, dst_rv,
                            req2veh_edge_attr[:, 0], req2veh_edge_attr[:, 1],
                            cst)
    gmax = jnp.max(logits)
    gmax16 = jnp.full((16,), gmax, jnp.float32)
    vfA = veh_feat[:, :32]
    vfB = veh_feat[:, 32:]
    nsumA, nsumB, dsum1, rvcnt1 = _run_rv_acc(
        vfA, vfB, logits, src_rv, dst_rv, gmax16, R)
    nsum = jnp.concatenate([nsumA, nsumB], axis=1)
    dsum = dsum1.reshape(-1, 1)
    rvcnt = rvcnt1.reshape(-1, 1)

    Wn1 = W_trip_n[:8]
    Wn2 = W_trip_n[8:40]
    return _run_head(requests_x, esum, ecnt, nsum, dsum, rvcnt,
                     W_req, b_req, Wn1, Wn2, b_trip_n,
                     W_act1, b_act1, W_act2, b_act2, W_act3, b_act3)
